# baseline (speedup 1.0000x reference)
import jax
import jax.numpy as jnp
from jax import lax
from jax.experimental import pallas as pl
from jax.experimental.pallas import tpu as pltpu
import numpy as np

B = 64
S = 8
V = 100000
VB = 12800
NBLK = (V + VB - 1) // VB
LCH = VB // 128
NSTRIP = B // 8

_KS0 = np.uint32(0)
_KS1 = np.uint32(42)
_KS2 = np.uint32(42) ^ np.uint32(0x1BD11BDA)
_R0 = (13, 15, 26, 6)
_R1 = (17, 29, 16, 24)
_NEG_INF = np.float32(-np.inf)
_IMAX = np.int32(2**31 - 1)


def _threefry_bits(cnt):
    x0 = jnp.zeros_like(cnt)
    x1 = cnt + _KS1

    def rnd(x0, x1, r):
        x0 = x0 + x1
        x1 = (x1 << np.uint32(r)) | (x1 >> np.uint32(32 - r))
        return x0, x1 ^ x0

    sched = ((_R0, _KS1, _KS2, 1), (_R1, _KS2, _KS0, 2), (_R0, _KS0, _KS1, 3),
             (_R1, _KS1, _KS2, 4), (_R0, _KS2, _KS0, 5))
    for rots, a0, a1, c in sched:
        for r in rots:
            x0, x1 = rnd(x0, x1, r)
        x0 = x0 + a0
        x1 = x1 + (a1 + np.uint32(c))
    return x0 ^ x1


def _gumbel_from_bits(bits):
    fb = (bits >> np.uint32(9)) | np.uint32(0x3F800000)
    u = lax.bitcast_convert_type(fb, jnp.float32) - jnp.float32(1.0)
    return -jnp.log(-jnp.log(u))


def _compose_kernel(x_ref, mask_ref, final_ref, ids_ref, m_scr, i_scr):
    j = pl.program_id(0)
    lane = lax.broadcasted_iota(jnp.int32, (8, 128), 1)
    sub = lax.broadcasted_iota(jnp.int32, (8, 128), 0)

    for s in range(NSTRIP):
        row = sub + (s * 8)

        def body(l, carry):
            acc_y, acc_c = carry
            x = x_ref[pl.ds(s * 8, 8), pl.ds(l * 128, 128)]
            mb = mask_ref[0, pl.ds(l * 128, 128)][None, :]
            fchunk = x + mb
            final_ref[pl.ds(s * 8, 8), pl.ds(l * 128, 128)] = fchunk
            col = lane + (j * VB + l * 128)
            cnt = (row * V + col).astype(jnp.uint32)
            g = _gumbel_from_bits(_threefry_bits(cnt))
            y = jnp.where(col < V, fchunk + g, _NEG_INF)
            upd = y > acc_y
            return (jnp.where(upd, y, acc_y), jnp.where(upd, col, acc_c))

        acc_y0 = jnp.full((8, 128), _NEG_INF, jnp.float32)
        acc_c0 = jnp.zeros((8, 128), jnp.int32)
        acc_y, acc_c = lax.fori_loop(0, LCH, body, (acc_y0, acc_c0))

        m8 = jnp.max(acc_y, axis=1)
        cand = jnp.where(acc_y == m8[:, None], acc_c, _IMAX)
        idx8 = jnp.min(cand, axis=1)

        @pl.when(j == 0)
        def _():
            m_scr[pl.ds(s * 8, 8), :] = m8[:, None]
            i_scr[pl.ds(s * 8, 8), :] = idx8[:, None]

        @pl.when(j > 0)
        def _():
            old_m = m_scr[pl.ds(s * 8, 8), :]
            old_i = i_scr[pl.ds(s * 8, 8), :]
            better = m8[:, None] > old_m
            m_scr[pl.ds(s * 8, 8), :] = jnp.where(better, m8[:, None], old_m)
            i_scr[pl.ds(s * 8, 8), :] = jnp.where(better, idx8[:, None], old_i)

    @pl.when(j == NBLK - 1)
    def _():
        ids_ref[...] = i_scr[...]


def kernel(logits, prediction_mask):
    last = logits[:, -1, :]
    mask2 = prediction_mask[None, :]
    final, ids2d = pl.pallas_call(
        _compose_kernel,
        grid=(NBLK,),
        in_specs=[
            pl.BlockSpec((B, VB), lambda j: (0, j)),
            pl.BlockSpec((1, VB), lambda j: (0, j)),
        ],
        out_specs=[
            pl.BlockSpec((B, VB), lambda j: (0, j)),
            pl.BlockSpec((B, 1), lambda j: (0, 0)),
        ],
        out_shape=[
            jax.ShapeDtypeStruct((B, V), jnp.float32),
            jax.ShapeDtypeStruct((B, 1), jnp.int32),
        ],
        scratch_shapes=[
            pltpu.VMEM((B, 1), jnp.float32),
            pltpu.VMEM((B, 1), jnp.int32),
        ],
    )(last, mask2)
    return ids2d[:, 0], final


# speedup vs baseline: 2.9025x; 2.9025x over previous
import jax
import jax.numpy as jnp
from jax import lax
from jax.experimental import pallas as pl
from jax.experimental.pallas import tpu as pltpu
import numpy as np

B = 64
S = 8
V = 100000
VB = 12800
NBLK = (V + VB - 1) // VB
LCH = VB // 128

_KS0 = np.uint32(0)
_KS1 = np.uint32(42)
_KS2 = np.uint32(42) ^ np.uint32(0x1BD11BDA)
_R0 = (13, 15, 26, 6)
_R1 = (17, 29, 16, 24)
_NEG_INF = np.float32(-np.inf)
_IMAX = np.int32(2**31 - 1)


def _threefry_bits(cnt):
    x0 = jnp.zeros_like(cnt)
    x1 = cnt + _KS1

    def rnd(x0, x1, r):
        x0 = x0 + x1
        x1 = (x1 << np.uint32(r)) | (x1 >> np.uint32(32 - r))
        return x0, x1 ^ x0

    sched = ((_R0, _KS1, _KS2, 1), (_R1, _KS2, _KS0, 2), (_R0, _KS0, _KS1, 3),
             (_R1, _KS1, _KS2, 4), (_R0, _KS2, _KS0, 5))
    for rots, a0, a1, c in sched:
        for r in rots:
            x0, x1 = rnd(x0, x1, r)
        x0 = x0 + a0
        x1 = x1 + (a1 + np.uint32(c))
    return x0 ^ x1


def _gumbel_from_bits(bits):
    fb = (bits >> np.uint32(9)) | np.uint32(0x3F800000)
    u = lax.bitcast_convert_type(fb, jnp.float32) - jnp.float32(1.0)
    return -jnp.log(-jnp.log(u))


def _compose_kernel(x_ref, mask_ref, final_ref, ids_ref, ay_scr, ac_scr):
    j = pl.program_id(0)
    lane = lax.broadcasted_iota(jnp.int32, (B, 128), 1)
    row = lax.broadcasted_iota(jnp.int32, (B, 128), 0)
    base = row * V + lane

    @pl.when(j == 0)
    def _():
        ay_scr[...] = jnp.full((B, 128), _NEG_INF, jnp.float32)
        ac_scr[...] = jnp.zeros((B, 128), jnp.int32)

    def body(l, carry):
        acc_y, acc_c = carry
        off = j * VB + l * 128
        x = x_ref[:, pl.ds(l * 128, 128)]
        mb = mask_ref[:, pl.ds(l * 128, 128)]
        fchunk = x + mb
        final_ref[:, pl.ds(l * 128, 128)] = fchunk
        cnt = (base + off).astype(jnp.uint32)
        g = _gumbel_from_bits(_threefry_bits(cnt))
        col = lane + off
        y = jnp.where(col < V, fchunk + g, _NEG_INF)
        upd = y > acc_y
        return (jnp.where(upd, y, acc_y), jnp.where(upd, col, acc_c))

    acc_y, acc_c = lax.fori_loop(0, LCH, body, (ay_scr[...], ac_scr[...]))
    ay_scr[...] = acc_y
    ac_scr[...] = acc_c

    @pl.when(j == NBLK - 1)
    def _():
        m = jnp.max(acc_y, axis=1)
        cand = jnp.where(acc_y == m[:, None], acc_c, _IMAX)
        ids_ref[...] = jnp.min(cand, axis=1)[:, None]


def kernel(logits, prediction_mask):
    last = logits[:, -1, :]
    mask2 = prediction_mask[None, :]
    final, ids2d = pl.pallas_call(
        _compose_kernel,
        grid=(NBLK,),
        in_specs=[
            pl.BlockSpec((B, VB), lambda j: (0, j)),
            pl.BlockSpec((1, VB), lambda j: (0, j)),
        ],
        out_specs=[
            pl.BlockSpec((B, VB), lambda j: (0, j)),
            pl.BlockSpec((B, 1), lambda j: (0, 0)),
        ],
        out_shape=[
            jax.ShapeDtypeStruct((B, V), jnp.float32),
            jax.ShapeDtypeStruct((B, 1), jnp.int32),
        ],
        scratch_shapes=[
            pltpu.VMEM((B, 128), jnp.float32),
            pltpu.VMEM((B, 128), jnp.int32),
        ],
    )(last, mask2)
    return ids2d[:, 0], final


# speedup vs baseline: 3.3036x; 1.1382x over previous
import jax
import jax.numpy as jnp
from jax import lax
from jax.experimental import pallas as pl
from jax.experimental.pallas import tpu as pltpu
import numpy as np

B = 64
S = 8
V = 100000
VB = 12800
NBLK = (V + VB - 1) // VB
LCH = VB // 128

_KS0 = np.uint32(0)
_KS1 = np.uint32(42)
_KS2 = np.uint32(42) ^ np.uint32(0x1BD11BDA)
_R0 = (13, 15, 26, 6)
_R1 = (17, 29, 16, 24)
_NEG_INF = np.float32(-np.inf)
_IMAX = np.int32(2**31 - 1)


def _threefry_bits(cnt):
    x0 = jnp.zeros_like(cnt)
    x1 = cnt + _KS1

    def rnd(x0, x1, r):
        x0 = x0 + x1
        x1 = (x1 << np.uint32(r)) | (x1 >> np.uint32(32 - r))
        return x0, x1 ^ x0

    sched = ((_R0, _KS1, _KS2, 1), (_R1, _KS2, _KS0, 2), (_R0, _KS0, _KS1, 3),
             (_R1, _KS1, _KS2, 4), (_R0, _KS2, _KS0, 5))
    for rots, a0, a1, c in sched:
        for r in rots:
            x0, x1 = rnd(x0, x1, r)
        x0 = x0 + a0
        x1 = x1 + (a1 + np.uint32(c))
    return x0 ^ x1


def _gumbel_from_bits(bits):
    fb = (bits >> np.uint32(9)) | np.uint32(0x3F800000)
    u = lax.bitcast_convert_type(fb, jnp.float32) - jnp.float32(1.0)
    return -jnp.log(-jnp.log(u))


def _compose_kernel(x_ref, mask_ref, final_ref, ids_ref, ay_scr, ac_scr):
    j = pl.program_id(0)

    @pl.when(j == 0)
    def _():
        ay_scr[...] = jnp.full((B, 128), _NEG_INF, jnp.float32)
        ac_scr[...] = jnp.zeros((B, 128), jnp.int32)

    lane = lax.broadcasted_iota(jnp.int32, (B, 128), 1)
    row = lax.broadcasted_iota(jnp.int32, (B, 128), 0)
    cnt0 = (row * V + lane + j * VB).astype(jnp.uint32)

    def body(l, carry):
        cnt, acc_y, acc_c = carry
        cnt_next = cnt + np.uint32(128)
        sl = pl.ds(l * 128, 128)
        x = x_ref[:, sl]
        mb = mask_ref[:, sl]
        final_ref[:, sl] = x + mb
        g = _gumbel_from_bits(_threefry_bits(cnt))
        y = final_ref[:, sl] + g
        upd = y > acc_y
        ci = j * LCH + l
        return (cnt_next,
                jnp.where(upd, y, acc_y),
                jnp.where(upd, ci, acc_c))

    _, acc_y, acc_c = lax.fori_loop(
        0, LCH, body, (cnt0, ay_scr[...], ac_scr[...]), unroll=2)
    ay_scr[...] = acc_y
    ac_scr[...] = acc_c

    @pl.when(j == NBLK - 1)
    def _():
        m = jnp.max(acc_y, axis=1)
        col = acc_c * 128 + lane
        cand = jnp.where(acc_y == m[:, None], col, _IMAX)
        ids_ref[...] = jnp.min(cand, axis=1)[:, None]


def kernel(logits, prediction_mask):
    last = logits[:, -1, :]
    mask2 = jnp.pad(prediction_mask, (0, NBLK * VB - V),
                    constant_values=-np.inf)[None, :]
    final, ids2d = pl.pallas_call(
        _compose_kernel,
        grid=(NBLK,),
        in_specs=[
            pl.BlockSpec((B, VB), lambda j: (0, j)),
            pl.BlockSpec((1, VB), lambda j: (0, j)),
        ],
        out_specs=[
            pl.BlockSpec((B, VB), lambda j: (0, j)),
            pl.BlockSpec((B, 1), lambda j: (0, 0)),
        ],
        out_shape=[
            jax.ShapeDtypeStruct((B, V), jnp.float32),
            jax.ShapeDtypeStruct((B, 1), jnp.int32),
        ],
        scratch_shapes=[
            pltpu.VMEM((B, 128), jnp.float32),
            pltpu.VMEM((B, 128), jnp.int32),
        ],
    )(last, mask2)
    return ids2d[:, 0], final


# speedup vs baseline: 3.3226x; 1.0058x over previous
import jax
import jax.numpy as jnp
from jax import lax
from jax.experimental import pallas as pl
from jax.experimental.pallas import tpu as pltpu
import numpy as np

B = 64
S = 8
V = 100000
VB = 12800
NBLK = (V + VB - 1) // VB
LCH = VB // 128

_KS0 = np.uint32(0)
_KS1 = np.uint32(42)
_KS2 = np.uint32(42) ^ np.uint32(0x1BD11BDA)
_R0 = (13, 15, 26, 6)
_R1 = (17, 29, 16, 24)
_NEG_INF = np.float32(-np.inf)
_IMAX = np.int32(2**31 - 1)


def _threefry_bits(cnt):
    x0 = jnp.zeros_like(cnt)
    x1 = cnt + _KS1

    def rnd(x0, x1, r):
        x0 = x0 + x1
        x1 = (x1 << np.uint32(r)) | (x1 >> np.uint32(32 - r))
        return x0, x1 ^ x0

    sched = ((_R0, _KS1, _KS2, 1), (_R1, _KS2, _KS0, 2), (_R0, _KS0, _KS1, 3),
             (_R1, _KS1, _KS2, 4), (_R0, _KS2, _KS0, 5))
    for rots, a0, a1, c in sched:
        for r in rots:
            x0, x1 = rnd(x0, x1, r)
        x0 = x0 + a0
        x1 = x1 + (a1 + np.uint32(c))
    return x0 ^ x1


def _gumbel_from_bits(bits):
    fb = (bits >> np.uint32(9)) | np.uint32(0x3F800000)
    u = lax.bitcast_convert_type(fb, jnp.float32) - jnp.float32(1.0)
    return -jnp.log(-jnp.log(u))


def _compose_kernel(x_ref, mask_ref, final_ref, ids_ref, ay_scr, ac_scr):
    j = pl.program_id(0)

    @pl.when(j == 0)
    def _():
        ay_scr[...] = jnp.full((B, 128), _NEG_INF, jnp.float32)
        ac_scr[...] = jnp.zeros((B, 128), jnp.int32)

    lane = lax.broadcasted_iota(jnp.int32, (B, 128), 1)
    row = lax.broadcasted_iota(jnp.int32, (B, 128), 0)
    cnt0 = (row * V + lane + j * VB).astype(jnp.uint32)

    def body(l, carry):
        cnt, acc_y, acc_c = carry
        cnt_next = cnt + np.uint32(128)
        sl = pl.ds(l * 128, 128)
        x = x_ref[:, sl]
        mb = mask_ref[:, sl]
        final_ref[:, sl] = x + mb
        g = _gumbel_from_bits(_threefry_bits(cnt))
        y = final_ref[:, sl] + g
        upd = y > acc_y
        ci = j * LCH + l
        return (cnt_next,
                jnp.where(upd, y, acc_y),
                jnp.where(upd, ci, acc_c))

    _, acc_y, acc_c = lax.fori_loop(
        0, LCH, body, (cnt0, ay_scr[...], ac_scr[...]), unroll=4)
    ay_scr[...] = acc_y
    ac_scr[...] = acc_c

    @pl.when(j == NBLK - 1)
    def _():
        m = jnp.max(acc_y, axis=1)
        col = acc_c * 128 + lane
        cand = jnp.where(acc_y == m[:, None], col, _IMAX)
        ids_ref[...] = jnp.min(cand, axis=1)[:, None]


def kernel(logits, prediction_mask):
    last = logits[:, -1, :]
    mask2 = jnp.pad(prediction_mask, (0, NBLK * VB - V),
                    constant_values=-np.inf)[None, :]
    final, ids2d = pl.pallas_call(
        _compose_kernel,
        grid=(NBLK,),
        in_specs=[
            pl.BlockSpec((B, VB), lambda j: (0, j)),
            pl.BlockSpec((1, VB), lambda j: (0, j)),
        ],
        out_specs=[
            pl.BlockSpec((B, VB), lambda j: (0, j)),
            pl.BlockSpec((B, 1), lambda j: (0, 0)),
        ],
        out_shape=[
            jax.ShapeDtypeStruct((B, V), jnp.float32),
            jax.ShapeDtypeStruct((B, 1), jnp.int32),
        ],
        scratch_shapes=[
            pltpu.VMEM((B, 128), jnp.float32),
            pltpu.VMEM((B, 128), jnp.int32),
        ],
    )(last, mask2)
    return ids2d[:, 0], final


# speedup vs baseline: 4.7548x; 1.4310x over previous
import jax
import jax.numpy as jnp
from jax import lax
from jax.experimental import pallas as pl
from jax.experimental.pallas import tpu as pltpu
import numpy as np

B = 64
S = 8
V = 100000
VB = 6400
NBLK = (V + VB - 1) // VB
LCH = VB // 128

_KS0 = np.uint32(0)
_KS1 = np.uint32(42)
_KS2 = np.uint32(42) ^ np.uint32(0x1BD11BDA)
_R0 = (13, 15, 26, 6)
_R1 = (17, 29, 16, 24)
_NEG_INF = np.float32(-np.inf)
_IMAX = np.int32(2**31 - 1)


def _threefry_bits(cnt):
    x0 = jnp.zeros_like(cnt)
    x1 = cnt + _KS1

    def rnd(x0, x1, r):
        x0 = x0 + x1
        x1 = (x1 << np.uint32(r)) | (x1 >> np.uint32(32 - r))
        return x0, x1 ^ x0

    sched = ((_R0, _KS1, _KS2, 1), (_R1, _KS2, _KS0, 2), (_R0, _KS0, _KS1, 3),
             (_R1, _KS1, _KS2, 4), (_R0, _KS2, _KS0, 5))
    for rots, a0, a1, c in sched:
        for r in rots:
            x0, x1 = rnd(x0, x1, r)
        x0 = x0 + a0
        x1 = x1 + (a1 + np.uint32(c))
    return x0 ^ x1


def _gumbel_from_bits(bits):
    fb = (bits >> np.uint32(9)) | np.uint32(0x3F800000)
    u = lax.bitcast_convert_type(fb, jnp.float32) - jnp.float32(1.0)
    return -jnp.log(-jnp.log(u))


def _compose_kernel(x_ref, mask_ref, final_ref, ids_ref, ay_scr, ac_scr):
    j = pl.program_id(0)

    @pl.when(j == 0)
    def _():
        ay_scr[...] = jnp.full((B, 128), _NEG_INF, jnp.float32)
        ac_scr[...] = jnp.zeros((B, 128), jnp.int32)

    lane = lax.broadcasted_iota(jnp.int32, (B, 128), 1)
    row = lax.broadcasted_iota(jnp.int32, (B, 128), 0)
    cnt0 = (row * V + lane + j * VB).astype(jnp.uint32)

    def body(l, carry):
        cnt, acc_y, acc_c = carry
        cnt_next = cnt + np.uint32(128)
        sl = pl.ds(l * 128, 128)
        x = x_ref[:, S - 1, sl]
        mb = mask_ref[:, sl]
        final_ref[:, sl] = x + mb
        g = _gumbel_from_bits(_threefry_bits(cnt))
        y = final_ref[:, sl] + g
        upd = y > acc_y
        ci = j * LCH + l
        return (cnt_next,
                jnp.where(upd, y, acc_y),
                jnp.where(upd, ci, acc_c))

    _, acc_y, acc_c = lax.fori_loop(
        0, LCH, body, (cnt0, ay_scr[...], ac_scr[...]), unroll=2)
    ay_scr[...] = acc_y
    ac_scr[...] = acc_c

    @pl.when(j == NBLK - 1)
    def _():
        m = jnp.max(acc_y, axis=1)
        col = acc_c * 128 + lane
        cand = jnp.where(acc_y == m[:, None], col, _IMAX)
        ids_ref[...] = jnp.min(cand, axis=1)[:, None]


def kernel(logits, prediction_mask):
    mask2 = jnp.pad(prediction_mask, (0, NBLK * VB - V),
                    constant_values=-np.inf)[None, :]
    final, ids2d = pl.pallas_call(
        _compose_kernel,
        grid=(NBLK,),
        in_specs=[
            pl.BlockSpec((B, S, VB), lambda j: (0, 0, j)),
            pl.BlockSpec((1, VB), lambda j: (0, j)),
        ],
        out_specs=[
            pl.BlockSpec((B, VB), lambda j: (0, j)),
            pl.BlockSpec((B, 1), lambda j: (0, 0)),
        ],
        out_shape=[
            jax.ShapeDtypeStruct((B, V), jnp.float32),
            jax.ShapeDtypeStruct((B, 1), jnp.int32),
        ],
        scratch_shapes=[
            pltpu.VMEM((B, 128), jnp.float32),
            pltpu.VMEM((B, 128), jnp.int32),
        ],
    )(logits, mask2)
    return ids2d[:, 0], final


# speedup vs baseline: 5.0986x; 1.0723x over previous
import jax
import jax.numpy as jnp
from jax import lax
from jax.experimental import pallas as pl
from jax.experimental.pallas import tpu as pltpu
import numpy as np

B = 64
S = 8
V = 100000
VB = 6400
NBLK = (V + VB - 1) // VB
LCH = VB // 128

_KS0 = np.uint32(0)
_KS1 = np.uint32(42)
_KS2 = np.uint32(42) ^ np.uint32(0x1BD11BDA)
_R0 = (13, 15, 26, 6)
_R1 = (17, 29, 16, 24)
_NEG_INF = np.float32(-np.inf)
_IMAX = np.int32(2**31 - 1)


def _threefry_bits(cnt):
    x0 = jnp.zeros_like(cnt)
    x1 = cnt + _KS1

    def rnd(x0, x1, r):
        x0 = x0 + x1
        x1 = (x1 << np.uint32(r)) | (x1 >> np.uint32(32 - r))
        return x0, x1 ^ x0

    sched = ((_R0, _KS1, _KS2, 1), (_R1, _KS2, _KS0, 2), (_R0, _KS0, _KS1, 3),
             (_R1, _KS1, _KS2, 4), (_R0, _KS2, _KS0, 5))
    for rots, a0, a1, c in sched:
        for r in rots:
            x0, x1 = rnd(x0, x1, r)
        x0 = x0 + a0
        x1 = x1 + (a1 + np.uint32(c))
    return x0 ^ x1


def _gumbel_from_bits(bits):
    fb = (bits >> np.uint32(9)) | np.uint32(0x3F800000)
    u = lax.bitcast_convert_type(fb, jnp.float32) - jnp.float32(1.0)
    return -jnp.log(-jnp.log(u))


def _compose_kernel(x_ref, mask_ref, final_ref, ids_ref, ay_scr, ac_scr):
    j = pl.program_id(0)

    @pl.when(j == 0)
    def _():
        ay_scr[...] = jnp.full((B, 128), _NEG_INF, jnp.float32)
        ac_scr[...] = jnp.zeros((B, 128), jnp.int32)

    lane = lax.broadcasted_iota(jnp.int32, (B, 128), 1)
    row = lax.broadcasted_iota(jnp.int32, (B, 128), 0)
    cnt0 = (row * V + lane + j * VB).astype(jnp.uint32)

    def body(l, carry):
        cnt, acc_y, acc_c = carry
        cnt_next = cnt + np.uint32(128)
        sl = pl.ds(l * 128, 128)
        x = x_ref[:, S - 1, sl]
        mb = mask_ref[:, sl]
        final_ref[:, sl] = x + mb
        g = _gumbel_from_bits(_threefry_bits(cnt))
        y = final_ref[:, sl] + g
        upd = y > acc_y
        ci = j * LCH + l
        return (cnt_next,
                jnp.where(upd, y, acc_y),
                jnp.where(upd, ci, acc_c))

    _, acc_y, acc_c = lax.fori_loop(
        0, LCH, body, (cnt0, ay_scr[...], ac_scr[...]), unroll=4)
    ay_scr[...] = acc_y
    ac_scr[...] = acc_c

    @pl.when(j == NBLK - 1)
    def _():
        m = jnp.max(acc_y, axis=1)
        col = acc_c * 128 + lane
        cand = jnp.where(acc_y == m[:, None], col, _IMAX)
        ids_ref[...] = jnp.min(cand, axis=1)[:, None]


def kernel(logits, prediction_mask):
    mask2 = jnp.pad(prediction_mask, (0, NBLK * VB - V),
                    constant_values=-np.inf)[None, :]
    final, ids2d = pl.pallas_call(
        _compose_kernel,
        grid=(NBLK,),
        in_specs=[
            pl.BlockSpec((B, S, VB), lambda j: (0, 0, j)),
            pl.BlockSpec((1, VB), lambda j: (0, j)),
        ],
        out_specs=[
            pl.BlockSpec((B, VB), lambda j: (0, j)),
            pl.BlockSpec((B, 1), lambda j: (0, 0)),
        ],
        out_shape=[
            jax.ShapeDtypeStruct((B, V), jnp.float32),
            jax.ShapeDtypeStruct((B, 1), jnp.int32),
        ],
        scratch_shapes=[
            pltpu.VMEM((B, 128), jnp.float32),
            pltpu.VMEM((B, 128), jnp.int32),
        ],
    )(logits, mask2)
    return ids2d[:, 0], final


# speedup vs baseline: 5.3281x; 1.0450x over previous
import jax
import jax.numpy as jnp
from jax import lax
from jax.experimental import pallas as pl
from jax.experimental.pallas import tpu as pltpu
import numpy as np

B = 64
S = 8
V = 100000
VB = 6400
NBLK = (V + VB - 1) // VB
LCH = VB // 128

_KS0 = np.uint32(0)
_KS1 = np.uint32(42)
_KS2 = np.uint32(42) ^ np.uint32(0x1BD11BDA)
_R0 = (13, 15, 26, 6)
_R1 = (17, 29, 16, 24)
_NEG_INF = np.float32(-np.inf)
_IMAX = np.int32(2**31 - 1)


def _threefry_bits(cnt):
    x0 = jnp.zeros_like(cnt)
    x1 = cnt + _KS1

    def rnd(x0, x1, r):
        x0 = x0 + x1
        x1 = (x1 << np.uint32(r)) | (x1 >> np.uint32(32 - r))
        return x0, x1 ^ x0

    sched = ((_R0, _KS1, _KS2, 1), (_R1, _KS2, _KS0, 2), (_R0, _KS0, _KS1, 3),
             (_R1, _KS1, _KS2, 4), (_R0, _KS2, _KS0, 5))
    for rots, a0, a1, c in sched:
        for r in rots:
            x0, x1 = rnd(x0, x1, r)
        x0 = x0 + a0
        x1 = x1 + (a1 + np.uint32(c))
    return x0 ^ x1


def _gumbel_from_bits(bits):
    fb = (bits >> np.uint32(9)) | np.uint32(0x3F800000)
    u = lax.bitcast_convert_type(fb, jnp.float32) - jnp.float32(1.0)
    return -jnp.log(-jnp.log(u))


def _compose_kernel(x_ref, mask_ref, final_ref, ids_ref, ay_scr, ac_scr):
    j = pl.program_id(0)

    @pl.when(j == 0)
    def _():
        ay_scr[...] = jnp.full((B, 128), _NEG_INF, jnp.float32)
        ac_scr[...] = jnp.zeros((B, 128), jnp.int32)

    lane = lax.broadcasted_iota(jnp.int32, (B, 128), 1)
    row = lax.broadcasted_iota(jnp.int32, (B, 128), 0)
    cnt0 = (row * V + lane + j * VB).astype(jnp.uint32)

    def body(l, carry):
        cnt, acc_y, acc_c = carry
        cnt_next = cnt + np.uint32(128)
        sl = pl.ds(l * 128, 128)
        x = x_ref[:, S - 1, sl]
        mb = mask_ref[:, sl]
        final_ref[:, sl] = x + mb
        g = _gumbel_from_bits(_threefry_bits(cnt))
        y = final_ref[:, sl] + g
        upd = y > acc_y
        ci = j * LCH + l
        return (cnt_next,
                jnp.where(upd, y, acc_y),
                jnp.where(upd, ci, acc_c))

    _, acc_y, acc_c = lax.fori_loop(
        0, LCH, body, (cnt0, ay_scr[...], ac_scr[...]), unroll=8)
    ay_scr[...] = acc_y
    ac_scr[...] = acc_c

    @pl.when(j == NBLK - 1)
    def _():
        m = jnp.max(acc_y, axis=1)
        col = acc_c * 128 + lane
        cand = jnp.where(acc_y == m[:, None], col, _IMAX)
        ids_ref[...] = jnp.min(cand, axis=1)[:, None]


def kernel(logits, prediction_mask):
    mask2 = jnp.pad(prediction_mask, (0, NBLK * VB - V),
                    constant_values=-np.inf)[None, :]
    final, ids2d = pl.pallas_call(
        _compose_kernel,
        grid=(NBLK,),
        in_specs=[
            pl.BlockSpec((B, S, VB), lambda j: (0, 0, j)),
            pl.BlockSpec((1, VB), lambda j: (0, j)),
        ],
        out_specs=[
            pl.BlockSpec((B, VB), lambda j: (0, j)),
            pl.BlockSpec((B, 1), lambda j: (0, 0)),
        ],
        out_shape=[
            jax.ShapeDtypeStruct((B, V), jnp.float32),
            jax.ShapeDtypeStruct((B, 1), jnp.int32),
        ],
        scratch_shapes=[
            pltpu.VMEM((B, 128), jnp.float32),
            pltpu.VMEM((B, 128), jnp.int32),
        ],
    )(logits, mask2)
    return ids2d[:, 0], final


# speedup vs baseline: 5.4100x; 1.0154x over previous
import jax
import jax.numpy as jnp
from jax import lax
from jax.experimental import pallas as pl
from jax.experimental.pallas import tpu as pltpu
import numpy as np

B = 64
S = 8
V = 100000
VB = 6400
NBLK = (V + VB - 1) // VB
LCH = VB // 128

_KS0 = np.uint32(0)
_KS1 = np.uint32(42)
_KS2 = np.uint32(42) ^ np.uint32(0x1BD11BDA)
_R0 = (13, 15, 26, 6)
_R1 = (17, 29, 16, 24)
_NEG_INF = np.float32(-np.inf)
_IMAX = np.int32(2**31 - 1)


def _threefry_bits(cnt):
    x0 = jnp.zeros_like(cnt)
    x1 = cnt + _KS1

    def rnd(x0, x1, r):
        x0 = x0 + x1
        x1 = (x1 << np.uint32(r)) | (x1 >> np.uint32(32 - r))
        return x0, x1 ^ x0

    sched = ((_R0, _KS1, _KS2, 1), (_R1, _KS2, _KS0, 2), (_R0, _KS0, _KS1, 3),
             (_R1, _KS1, _KS2, 4), (_R0, _KS2, _KS0, 5))
    for rots, a0, a1, c in sched:
        for r in rots:
            x0, x1 = rnd(x0, x1, r)
        x0 = x0 + a0
        x1 = x1 + (a1 + np.uint32(c))
    return x0 ^ x1


def _gumbel_from_bits(bits):
    fb = (bits >> np.uint32(9)) | np.uint32(0x3F800000)
    u = lax.bitcast_convert_type(fb, jnp.float32) - jnp.float32(1.0)
    return -jnp.log(-jnp.log(u))


def _compose_kernel(x_ref, mask_ref, final_ref, ids_ref, ay_scr, ac_scr):
    j = pl.program_id(0)

    @pl.when(j == 0)
    def _():
        ay_scr[...] = jnp.full((B, 128), _NEG_INF, jnp.float32)
        ac_scr[...] = jnp.zeros((B, 128), jnp.int32)

    lane = lax.broadcasted_iota(jnp.int32, (B, 128), 1)
    row = lax.broadcasted_iota(jnp.int32, (B, 128), 0)
    cnt0 = (row * V + lane + j * VB).astype(jnp.uint32)

    def body(l, carry):
        cnt, acc_y, acc_c = carry
        cnt_next = cnt + np.uint32(128)
        sl = pl.ds(l * 128, 128)
        x = x_ref[:, S - 1, sl]
        mb = mask_ref[:, sl]
        final_ref[:, sl] = x + mb
        g = _gumbel_from_bits(_threefry_bits(cnt))
        y = final_ref[:, sl] + g
        upd = y > acc_y
        ci = j * LCH + l
        return (cnt_next,
                jnp.where(upd, y, acc_y),
                jnp.where(upd, ci, acc_c))

    _, acc_y, acc_c = lax.fori_loop(
        0, LCH, body, (cnt0, ay_scr[...], ac_scr[...]), unroll=16)
    ay_scr[...] = acc_y
    ac_scr[...] = acc_c

    @pl.when(j == NBLK - 1)
    def _():
        m = jnp.max(acc_y, axis=1)
        col = acc_c * 128 + lane
        cand = jnp.where(acc_y == m[:, None], col, _IMAX)
        ids_ref[...] = jnp.min(cand, axis=1)[:, None]


def kernel(logits, prediction_mask):
    mask2 = jnp.pad(prediction_mask, (0, NBLK * VB - V),
                    constant_values=-np.inf)[None, :]
    final, ids2d = pl.pallas_call(
        _compose_kernel,
        grid=(NBLK,),
        in_specs=[
            pl.BlockSpec((B, S, VB), lambda j: (0, 0, j)),
            pl.BlockSpec((1, VB), lambda j: (0, j)),
        ],
        out_specs=[
            pl.BlockSpec((B, VB), lambda j: (0, j)),
            pl.BlockSpec((B, 1), lambda j: (0, 0)),
        ],
        out_shape=[
            jax.ShapeDtypeStruct((B, V), jnp.float32),
            jax.ShapeDtypeStruct((B, 1), jnp.int32),
        ],
        scratch_shapes=[
            pltpu.VMEM((B, 128), jnp.float32),
            pltpu.VMEM((B, 128), jnp.int32),
        ],
    )(logits, mask2)
    return ids2d[:, 0], final


# speedup vs baseline: 5.4453x; 1.0065x over previous
import jax
import jax.numpy as jnp
from jax import lax
from jax.experimental import pallas as pl
from jax.experimental.pallas import tpu as pltpu
import numpy as np

B = 64
S = 8
V = 100000
VB = 6400
NBLK = (V + VB - 1) // VB
LCH = VB // 128

_KS0 = np.uint32(0)
_KS1 = np.uint32(42)
_KS2 = np.uint32(42) ^ np.uint32(0x1BD11BDA)
_R0 = (13, 15, 26, 6)
_R1 = (17, 29, 16, 24)
_NEG_INF = np.float32(-np.inf)
_IMAX = np.int32(2**31 - 1)


def _threefry_bits(cnt):
    x0 = jnp.zeros_like(cnt)
    x1 = cnt + _KS1

    def rnd(x0, x1, r):
        x0 = x0 + x1
        x1 = (x1 << np.uint32(r)) | (x1 >> np.uint32(32 - r))
        return x0, x1 ^ x0

    sched = ((_R0, _KS1, _KS2, 1), (_R1, _KS2, _KS0, 2), (_R0, _KS0, _KS1, 3),
             (_R1, _KS1, _KS2, 4), (_R0, _KS2, _KS0, 5))
    for rots, a0, a1, c in sched:
        for r in rots:
            x0, x1 = rnd(x0, x1, r)
        x0 = x0 + a0
        x1 = x1 + (a1 + np.uint32(c))
    return x0 ^ x1


def _gumbel_from_bits(bits):
    fb = (bits >> np.uint32(9)) | np.uint32(0x3F800000)
    u = lax.bitcast_convert_type(fb, jnp.float32) - jnp.float32(1.0)
    return -jnp.log(-jnp.log(u))


def _compose_kernel(x_ref, mask_ref, final_ref, ids_ref, ay_scr, ac_scr):
    j = pl.program_id(0)

    @pl.when(j == 0)
    def _():
        ay_scr[...] = jnp.full((B, 128), _NEG_INF, jnp.float32)
        ac_scr[...] = jnp.zeros((B, 128), jnp.int32)

    lane = lax.broadcasted_iota(jnp.int32, (B, 128), 1)
    row = lax.broadcasted_iota(jnp.int32, (B, 128), 0)
    cnt0 = (row * V + lane + j * VB).astype(jnp.uint32)

    def body(l, carry):
        cnt, acc_y, acc_c = carry
        cnt_next = cnt + np.uint32(128)
        sl = pl.ds(l * 128, 128)
        x = x_ref[:, S - 1, sl]
        mb = mask_ref[:, sl]
        final_ref[:, sl] = x + mb
        g = _gumbel_from_bits(_threefry_bits(cnt))
        y = final_ref[:, sl] + g
        upd = y > acc_y
        ci = j * LCH + l
        return (cnt_next,
                jnp.where(upd, y, acc_y),
                jnp.where(upd, ci, acc_c))

    _, acc_y, acc_c = lax.fori_loop(
        0, LCH, body, (cnt0, ay_scr[...], ac_scr[...]), unroll=25)
    ay_scr[...] = acc_y
    ac_scr[...] = acc_c

    @pl.when(j == NBLK - 1)
    def _():
        m = jnp.max(acc_y, axis=1)
        col = acc_c * 128 + lane
        cand = jnp.where(acc_y == m[:, None], col, _IMAX)
        ids_ref[...] = jnp.min(cand, axis=1)[:, None]


def kernel(logits, prediction_mask):
    mask2 = jnp.pad(prediction_mask, (0, NBLK * VB - V),
                    constant_values=-np.inf)[None, :]
    final, ids2d = pl.pallas_call(
        _compose_kernel,
        grid=(NBLK,),
        in_specs=[
            pl.BlockSpec((B, S, VB), lambda j: (0, 0, j)),
            pl.BlockSpec((1, VB), lambda j: (0, j)),
        ],
        out_specs=[
            pl.BlockSpec((B, VB), lambda j: (0, j)),
            pl.BlockSpec((B, 1), lambda j: (0, 0)),
        ],
        out_shape=[
            jax.ShapeDtypeStruct((B, V), jnp.float32),
            jax.ShapeDtypeStruct((B, 1), jnp.int32),
        ],
        scratch_shapes=[
            pltpu.VMEM((B, 128), jnp.float32),
            pltpu.VMEM((B, 128), jnp.int32),
        ],
    )(logits, mask2)
    return ids2d[:, 0], final
